# baseline (speedup 1.0000x reference)
import jax
import jax.numpy as jnp
from jax.experimental import pallas as pl
from jax.experimental.pallas import tpu as pltpu

_KB = 2048
_NB = 4
_D = 128
_LANES = 128
_NIMG = 4
_PPI = 784


def _prep_block(k_ref, kb_ref, ksq_ref, row0, rows_left):
    kf = k_ref[row0:row0 + _KB, :]
    row_id = jax.lax.broadcasted_iota(jnp.int32, (_KB, _D), 0)
    kf = jnp.where(row_id < rows_left, kf, 0.0)
    kb_ref[...] = kf.astype(jnp.bfloat16)
    kk = (kf * kf).astype(jnp.bfloat16)
    ones = jnp.ones((8, _D), jnp.bfloat16)
    ksq = jax.lax.dot_general(
        ones, kk, (((1,), (1,)), ((), ())),
        preferred_element_type=jnp.float32)
    col_id = jax.lax.broadcasted_iota(jnp.int32, (8, _KB), 1)
    ksq_ref[...] = jnp.where(col_id < rows_left, ksq, 3.0e38)


def _tile_min(q_ref, kb_ref, ksq_ref):
    cand = jax.lax.dot_general(
        q_ref[...], kb_ref[...], (((1,), (1,)), ((), ())),
        preferred_element_type=jnp.float32)
    cand = cand + ksq_ref[0:1, :]
    m = cand[:, 0:_LANES]
    for c in range(1, _KB // _LANES):
        m = jnp.minimum(m, cand[:, c * _LANES:(c + 1) * _LANES])
    return m


def _stream_kernel(q_ref, k_ref, acc_ref, kba, ksqa, kbb, ksqb, kbc, ksqc, kbd, ksqd, *, kk_total):
    j = pl.program_id(0)
    base = j * (_NB * _KB)

    _prep_block(k_ref, kba, ksqa, 0, kk_total - base)
    _prep_block(k_ref, kbb, ksqb, _KB, kk_total - base - _KB)
    ma = _tile_min(q_ref, kba, ksqa)
    mb = _tile_min(q_ref, kbb, ksqb)
    _prep_block(k_ref, kbc, ksqc, 2 * _KB, kk_total - base - 2 * _KB)
    _prep_block(k_ref, kbd, ksqd, 3 * _KB, kk_total - base - 3 * _KB)
    mc = _tile_min(q_ref, kbc, ksqc)
    md = _tile_min(q_ref, kbd, ksqd)
    m = jnp.minimum(jnp.minimum(ma, mb), jnp.minimum(mc, md))

    @pl.when(j == 0)
    def _first():
        acc_ref[...] = m

    @pl.when(j > 0)
    def _rest():
        acc_ref[...] = jnp.minimum(acc_ref[...], m)


def _final_kernel(q_ref, acc_ref, ps_ref, im_ref):
    qf = q_ref[...].astype(jnp.float32)
    q_sq = 0.25 * jnp.sum(qf * qf, axis=1)
    mn = jnp.min(acc_ref[...], axis=1)
    d2 = jnp.maximum(q_sq + mn, 0.0)
    ps = jnp.sqrt(d2 + 1e-12)
    ps2 = ps.reshape(_NIMG, _PPI)
    ps_ref[...] = ps2.reshape(_NIMG, 1, _PPI)
    imax = jnp.max(ps2, axis=1).reshape(_NIMG, 1, 1)
    im_ref[...] = jnp.broadcast_to(imax, (_NIMG, 1, _LANES))


def kernel(queries, keys):
    Q, D = queries.shape
    K, _ = keys.shape
    step = _NB * _KB
    nj = (K + step - 1) // step

    qb = (-2.0 * queries).astype(jnp.bfloat16)

    minacc = pl.pallas_call(
        lambda qr, kr, ar, a1, a2, b1, b2, c1, c2, d1, d2: _stream_kernel(
            qr, kr, ar, a1, a2, b1, b2, c1, c2, d1, d2, kk_total=K),
        grid=(nj,),
        in_specs=[
            pl.BlockSpec((Q, _D), lambda j: (0, 0)),
            pl.BlockSpec((step, _D), lambda j: (j, 0)),
        ],
        out_specs=pl.BlockSpec((Q, _LANES), lambda j: (0, 0)),
        out_shape=jax.ShapeDtypeStruct((Q, _LANES), jnp.float32),
        scratch_shapes=[
            pltpu.VMEM((_KB, _D), jnp.bfloat16),
            pltpu.VMEM((8, _KB), jnp.float32),
            pltpu.VMEM((_KB, _D), jnp.bfloat16),
            pltpu.VMEM((8, _KB), jnp.float32),
            pltpu.VMEM((_KB, _D), jnp.bfloat16),
            pltpu.VMEM((8, _KB), jnp.float32),
            pltpu.VMEM((_KB, _D), jnp.bfloat16),
            pltpu.VMEM((8, _KB), jnp.float32),
        ],
    )(qb, keys)

    ps3, im3 = pl.pallas_call(
        _final_kernel,
        out_shape=[
            jax.ShapeDtypeStruct((_NIMG, 1, _PPI), jnp.float32),
            jax.ShapeDtypeStruct((_NIMG, 1, _LANES), jnp.float32),
        ],
    )(qb, minacc)

    patch_scores = ps3.reshape(-1)
    image_scores = im3[:, 0, 0].reshape(_NIMG)
    return image_scores, patch_scores


# speedup vs baseline: 1.0185x; 1.0185x over previous
import jax
import jax.numpy as jnp
from jax.experimental import pallas as pl
from jax.experimental.pallas import tpu as pltpu

_KB = 2048
_NB = 2
_D = 128
_LANES = 128
_NIMG = 4
_PPI = 784


def _prep_block(k_ref, kb_ref, ksq_ref, row0, rows_left):
    kf = k_ref[row0:row0 + _KB, :]
    row_id = jax.lax.broadcasted_iota(jnp.int32, (_KB, _D), 0)
    kf = jnp.where(row_id < rows_left, kf, 0.0)
    kb_ref[...] = kf.astype(jnp.bfloat16)
    kk = (kf * kf).astype(jnp.bfloat16)
    ones = jnp.ones((8, _D), jnp.bfloat16)
    ksq = jax.lax.dot_general(
        ones, kk, (((1,), (1,)), ((), ())),
        preferred_element_type=jnp.float32)
    col_id = jax.lax.broadcasted_iota(jnp.int32, (8, _KB), 1)
    ksq_ref[...] = jnp.where(col_id < rows_left, ksq, 3.0e38)


def _tile_min(q_ref, kb_ref, ksq_ref):
    cand = jax.lax.dot_general(
        q_ref[...], kb_ref[...], (((1,), (1,)), ((), ())),
        preferred_element_type=jnp.float32)
    cand = cand + ksq_ref[0:1, :]
    parts = [cand[:, c * _LANES:(c + 1) * _LANES]
             for c in range(_KB // _LANES)]
    while len(parts) > 1:
        parts = [jnp.minimum(parts[t], parts[t + 1])
                 for t in range(0, len(parts), 2)]
    return parts[0]


def _stream_kernel(q_ref, k_ref, acc_ref, kba, ksqa, kbb, ksqb, *, kk_total):
    j = pl.program_id(0)
    base = j * (_NB * _KB)

    _prep_block(k_ref, kba, ksqa, 0, kk_total - base)
    _prep_block(k_ref, kbb, ksqb, _KB, kk_total - base - _KB)
    ma = _tile_min(q_ref, kba, ksqa)
    mb = _tile_min(q_ref, kbb, ksqb)
    m = jnp.minimum(ma, mb)

    @pl.when(j == 0)
    def _first():
        acc_ref[...] = m

    @pl.when(j > 0)
    def _rest():
        acc_ref[...] = jnp.minimum(acc_ref[...], m)


def _final_kernel(q_ref, acc_ref, ps_ref, im_ref):
    qf = q_ref[...].astype(jnp.float32)
    q_sq = 0.25 * jnp.sum(qf * qf, axis=1)
    mn = jnp.min(acc_ref[...], axis=1)
    d2 = jnp.maximum(q_sq + mn, 0.0)
    ps = jnp.sqrt(d2 + 1e-12)
    ps2 = ps.reshape(_NIMG, _PPI)
    ps_ref[...] = ps2.reshape(_NIMG, 1, _PPI)
    imax = jnp.max(ps2, axis=1).reshape(_NIMG, 1, 1)
    im_ref[...] = jnp.broadcast_to(imax, (_NIMG, 1, _LANES))


def kernel(queries, keys):
    Q, D = queries.shape
    K, _ = keys.shape
    step = _NB * _KB
    nj = (K + step - 1) // step

    qb = (-2.0 * queries).astype(jnp.bfloat16)

    minacc = pl.pallas_call(
        lambda qr, kr, ar, a1, a2, b1, b2: _stream_kernel(
            qr, kr, ar, a1, a2, b1, b2, kk_total=K),
        grid=(nj,),
        in_specs=[
            pl.BlockSpec((Q, _D), lambda j: (0, 0)),
            pl.BlockSpec((step, _D), lambda j: (j, 0)),
        ],
        out_specs=pl.BlockSpec((Q, _LANES), lambda j: (0, 0)),
        out_shape=jax.ShapeDtypeStruct((Q, _LANES), jnp.float32),
        scratch_shapes=[
            pltpu.VMEM((_KB, _D), jnp.bfloat16),
            pltpu.VMEM((8, _KB), jnp.float32),
            pltpu.VMEM((_KB, _D), jnp.bfloat16),
            pltpu.VMEM((8, _KB), jnp.float32),
        ],
    )(qb, keys)

    ps3, im3 = pl.pallas_call(
        _final_kernel,
        out_shape=[
            jax.ShapeDtypeStruct((_NIMG, 1, _PPI), jnp.float32),
            jax.ShapeDtypeStruct((_NIMG, 1, _LANES), jnp.float32),
        ],
    )(qb, minacc)

    patch_scores = ps3.reshape(-1)
    image_scores = im3[:, 0, 0].reshape(_NIMG)
    return image_scores, patch_scores
